# baseline (speedup 1.0000x reference)
import functools
import jax
import jax.numpy as jnp
from jax import lax
from jax.experimental import pallas as pl
from jax.experimental.pallas import tpu as pltpu
from jax.experimental.pallas import tpu_sc as plsc

B = 4096
F = 26
VOCAB = 100000
D = 32

NC = 2
NS = 16
NW = NC * NS
BPW = B // NW
NG = B // 16

VMAIN = 99968
H0 = 50048
H1 = VMAIN - H0
HBUF = H0
TAILW = 128


def _stream_body(xi_hbm, emb_hbm, tails_hbm, parts_hbm,
                 idx_v, buf, tails_v, s_acc, q_acc, sem0, sem1, sem_t,
                 sem_x0, sem_x1):
    c = lax.axis_index("c")
    s = lax.axis_index("s")
    w = s * NC + c

    zero = jnp.zeros((16,), jnp.float32)

    def clear(i, carry):
        s_acc[pl.ds(i * 16, 16)] = zero
        q_acc[pl.ds(i * 16, 16)] = zero
        return carry

    lax.fori_loop(0, NG, clear, 0)

    tail_copies = []
    for f in range(F):
        cp = pltpu.make_async_copy(
            tails_hbm.at[pl.ds((f * D + w) * TAILW, TAILW)],
            tails_v.at[pl.ds(f * TAILW, TAILW)], sem_t)
        tail_copies.append(cp)

    sems = (sem0, sem1)
    xsems = (sem_x0, sem_x1)
    HALF = ((0, H0), (H0, H1))

    def xi_fire(f, slot):
        pltpu.make_async_copy(
            xi_hbm.at[pl.ds(f * B, B)],
            idx_v.at[pl.ds(slot * B, B)], xsems[slot]).start()

    def xi_drain(slot):
        pltpu.make_async_copy(
            xi_hbm.at[pl.ds(0, B)],
            idx_v.at[pl.ds(slot * B, B)], xsems[slot]).wait()

    def fire(f, h):
        st, ln = HALF[h]
        pltpu.make_async_copy(
            emb_hbm.at[f * D + w, pl.ds(st, ln)],
            buf.at[pl.ds(h * HBUF, ln)], sems[h]).start()

    def drain(h):
        st, ln = HALF[h]
        pltpu.make_async_copy(
            emb_hbm.at[0, pl.ds(st, ln)],
            buf.at[pl.ds(h * HBUF, ln)], sems[h]).wait()

    def scan_half(f, h, xslot):
        base = HALF[h][0]
        boff = h * HBUF

        def scan(g, carry):
            p = pl.ds(g * 16, 16)
            v = idx_v[pl.ds(xslot * B + g * 16, 16)]
            voff = v - base
            m = jnp.logical_and(voff >= 0, voff < HBUF)
            vv = plsc.load_gather(
                buf, [boff + jnp.where(m, voff, 0)], mask=m)
            vsel = jnp.where(m, vv, 0.0)
            if h == 1:
                toff = v - VMAIN
                mt = toff >= 0
                tv = plsc.load_gather(
                    tails_v, [f * TAILW + jnp.where(mt, toff, 0)], mask=mt)
                vsel = vsel + jnp.where(mt, tv, 0.0)
            s_acc[p] = s_acc[p] + vsel
            q_acc[p] = q_acc[p] + vsel * vsel
            return carry

        lax.fori_loop(0, NG, scan, 0)

    fire(0, 0)
    xi_fire(0, 0)

    def step(fp, carry):
        for ff in (0, 1):
            f = fp * 2 + ff
            xi_drain(ff)

            @pl.when(f + 1 < F)
            def _():
                xi_fire(f + 1, 1 - ff)

            fire(f, 1)
            drain(0)
            scan_half(f, 0, ff)

            @pl.when(f + 1 < F)
            def _():
                fire(f + 1, 0)

            drain(1)
            scan_half(f, 1, ff)
        return carry

    for cp in tail_copies:
        cp.start()
    for cp in tail_copies:
        cp.wait()
    lax.fori_loop(0, F // 2, step, 0)

    def fin(g, carry):
        p = pl.ds(g * 16, 16)
        sa = s_acc[p]
        s_acc[p] = sa * sa - q_acc[p]
        return carry

    lax.fori_loop(0, NG, fin, 0)
    pltpu.sync_copy(s_acc, parts_hbm.at[pl.ds(w * B, B)])


def _reduce_body(parts_hbm, xi_hbm, lin_hbm, bias_hbm, out_hbm,
                 pv, idx_v, lidx, lin_v, out_v, bias_v, sem_p, sem_l):
    c = lax.axis_index("c")
    s = lax.axis_index("s")
    w = s * NC + c

    pltpu.sync_copy(bias_hbm, bias_v)

    stage = []
    for d in range(D):
        stage.append(pltpu.make_async_copy(
            parts_hbm.at[pl.ds(d * B + w * BPW, BPW)],
            pv.at[pl.ds(d * BPW, BPW)], sem_p))
    for f in range(F):
        stage.append(pltpu.make_async_copy(
            xi_hbm.at[pl.ds(f * B + w * BPW, BPW)],
            idx_v.at[pl.ds(f * BPW, BPW)], sem_p))
    for cp in stage:
        cp.start()
    for cp in stage:
        cp.wait()

    def lin_build(f, carry):
        def per_stripe(st, carry2):
            p = pl.ds(f * BPW + st * 16, 16)
            lidx[p] = f * VOCAB + idx_v[p]
            return carry2
        lax.fori_loop(0, BPW // 16, per_stripe, 0)
        return carry

    lax.fori_loop(0, F, lin_build, 0)
    lin_copies = []
    for j in range(F):
        cp = pltpu.make_async_copy(
            lin_hbm.at[lidx.at[pl.ds(j * 128, 128)]],
            lin_v.at[pl.ds(j * 128, 128)], sem_l)
        cp.start()
        lin_copies.append(cp)
    for cp in lin_copies:
        cp.wait()

    bias_vec = bias_v[...]

    def finish(g, carry):
        b0 = g * 16
        t = jnp.zeros((16,), jnp.float32)
        for d in range(D):
            t = t + pv[pl.ds(d * BPW + b0, 16)]
        lsum = bias_vec
        for f in range(F):
            lsum = lsum + lin_v[pl.ds(f * BPW + b0, 16)]
        out_v[pl.ds(b0, 16)] = 0.5 * t + lsum
        return carry

    lax.fori_loop(0, BPW // 16, finish, 0)
    pltpu.sync_copy(out_v, out_hbm.at[pl.ds(w * BPW, BPW)])


_mesh = plsc.VectorSubcoreMesh(core_axis_name="c", subcore_axis_name="s")

_stream = functools.partial(
    pl.kernel,
    mesh=_mesh,
    compiler_params=pltpu.CompilerParams(
        needs_layout_passes=False, use_tc_tiling_on_sc=True),
    out_type=jax.ShapeDtypeStruct((D * B,), jnp.float32),
    scratch_types=[
        pltpu.VMEM((2 * B,), jnp.int32),
        pltpu.VMEM((2 * HBUF,), jnp.float32),
        pltpu.VMEM((F * TAILW,), jnp.float32),
        pltpu.VMEM((B,), jnp.float32),
        pltpu.VMEM((B,), jnp.float32),
        pltpu.SemaphoreType.DMA,
        pltpu.SemaphoreType.DMA,
        pltpu.SemaphoreType.DMA,
        pltpu.SemaphoreType.DMA,
        pltpu.SemaphoreType.DMA,
    ],
)(_stream_body)

_reduce = functools.partial(
    pl.kernel,
    mesh=_mesh,
    compiler_params=pltpu.CompilerParams(
        needs_layout_passes=False, use_tc_tiling_on_sc=False),
    out_type=jax.ShapeDtypeStruct((B,), jnp.float32),
    scratch_types=[
        pltpu.VMEM((D * BPW,), jnp.float32),
        pltpu.VMEM((F * BPW,), jnp.int32),
        pltpu.VMEM((F * BPW,), jnp.int32),
        pltpu.VMEM((F * BPW,), jnp.float32),
        pltpu.VMEM((BPW,), jnp.float32),
        pltpu.VMEM((16,), jnp.float32),
        pltpu.SemaphoreType.DMA,
        pltpu.SemaphoreType.DMA,
    ],
)(_reduce_body)


@jax.jit
def kernel(Xi, W_lin, W_emb, bias):
    xi = Xi.reshape(B, F).astype(jnp.int32).T.reshape(F * B)
    emb = jnp.transpose(W_emb, (0, 2, 1)).reshape(F * D, VOCAB)
    tails = jnp.transpose(W_emb[:, VMAIN:, :], (0, 2, 1))
    tails = jnp.pad(tails, ((0, 0), (0, 0), (0, TAILW - (VOCAB - VMAIN))))
    tails = tails.reshape(F * D * TAILW)
    lin = W_lin.reshape(F * VOCAB)
    bias16 = jnp.broadcast_to(bias, (16,))
    parts = _stream(xi, emb, tails)
    return _reduce(parts, xi, lin, bias16)


# speedup vs baseline: 1.2072x; 1.2072x over previous
import functools
import jax
import jax.numpy as jnp
from jax import lax
from jax.experimental import pallas as pl
from jax.experimental.pallas import tpu as pltpu
from jax.experimental.pallas import tpu_sc as plsc

B = 4096
F = 26
VOCAB = 100000
D = 32

NC = 2
NS = 16
NW = NC * NS
BPW = B // NW
NG = B // 16

VMAIN = 99968
H0 = 50048
H1 = VMAIN - H0
HBUF = H0
TAILW = 128


def _stream_body(xi_hbm, emb_hbm, tails_hbm, parts_hbm,
                 idx_v, buf, tails_v, s_acc, q_acc, sem0, sem1, sem_t):
    c = lax.axis_index("c")
    s = lax.axis_index("s")
    w = s * NC + c

    zero = jnp.zeros((16,), jnp.float32)

    def clear(i, carry):
        s_acc[pl.ds(i * 16, 16)] = zero
        q_acc[pl.ds(i * 16, 16)] = zero
        return carry

    lax.fori_loop(0, NG, clear, 0)

    tail_copies = []
    for f in range(F):
        cp = pltpu.make_async_copy(
            tails_hbm.at[pl.ds((f * D + w) * TAILW, TAILW)],
            tails_v.at[pl.ds(f * TAILW, TAILW)], sem_t)
        tail_copies.append(cp)

    sems = (sem0, sem1)
    HALF = ((0, H0), (H0, H1))

    def fire(f, h):
        st, ln = HALF[h]
        pltpu.make_async_copy(
            emb_hbm.at[f * D + w, pl.ds(st, ln)],
            buf.at[pl.ds(h * HBUF, ln)], sems[h]).start()

    def drain(h):
        st, ln = HALF[h]
        pltpu.make_async_copy(
            emb_hbm.at[0, pl.ds(st, ln)],
            buf.at[pl.ds(h * HBUF, ln)], sems[h]).wait()

    def scan_half(f, h):
        base = HALF[h][0]
        boff = h * HBUF

        def scan(g, carry):
            p = pl.ds(g * 16, 16)
            v = idx_v[p]
            voff = v - base
            m = jnp.logical_and(voff >= 0, voff < HBUF)
            vv = plsc.load_gather(
                buf, [boff + jnp.where(m, voff, 0)], mask=m)
            vsel = jnp.where(m, vv, 0.0)
            if h == 1:
                toff = v - VMAIN
                mt = toff >= 0
                tv = plsc.load_gather(
                    tails_v, [f * TAILW + jnp.where(mt, toff, 0)], mask=mt)
                vsel = vsel + jnp.where(mt, tv, 0.0)
            s_acc[p] = s_acc[p] + vsel
            q_acc[p] = q_acc[p] + vsel * vsel
            return carry

        lax.fori_loop(0, NG, scan, 0)

    fire(0, 0)

    def step(f, carry):
        pltpu.sync_copy(xi_hbm.at[pl.ds(f * B, B)], idx_v)
        fire(f, 1)
        drain(0)
        scan_half(f, 0)

        @pl.when(f + 1 < F)
        def _():
            fire(f + 1, 0)

        drain(1)
        scan_half(f, 1)
        return carry

    for cp in tail_copies:
        cp.start()
    for cp in tail_copies:
        cp.wait()
    lax.fori_loop(0, F, step, 0)

    def fin(g, carry):
        p = pl.ds(g * 16, 16)
        sa = s_acc[p]
        s_acc[p] = sa * sa - q_acc[p]
        return carry

    lax.fori_loop(0, NG, fin, 0)
    pltpu.sync_copy(s_acc, parts_hbm.at[pl.ds(w * B, B)])


def _reduce_body(parts_hbm, xi_hbm, lin_hbm, bias_hbm, out_hbm,
                 pv, idx_v, lidx, lin_v, out_v, bias_v, sem_p, sem_l):
    c = lax.axis_index("c")
    s = lax.axis_index("s")
    w = s * NC + c

    pltpu.sync_copy(bias_hbm, bias_v)

    stage = []
    for d in range(D):
        stage.append(pltpu.make_async_copy(
            parts_hbm.at[pl.ds(d * B + w * BPW, BPW)],
            pv.at[pl.ds(d * BPW, BPW)], sem_p))
    for f in range(F):
        stage.append(pltpu.make_async_copy(
            xi_hbm.at[pl.ds(f * B + w * BPW, BPW)],
            idx_v.at[pl.ds(f * BPW, BPW)], sem_p))
    for cp in stage:
        cp.start()
    for cp in stage:
        cp.wait()

    def lin_build(f, carry):
        def per_stripe(st, carry2):
            p = pl.ds(f * BPW + st * 16, 16)
            lidx[p] = f * VOCAB + idx_v[p]
            return carry2
        lax.fori_loop(0, BPW // 16, per_stripe, 0)
        return carry

    lax.fori_loop(0, F, lin_build, 0)
    lin_copies = []
    for j in range(F):
        cp = pltpu.make_async_copy(
            lin_hbm.at[lidx.at[pl.ds(j * 128, 128)]],
            lin_v.at[pl.ds(j * 128, 128)], sem_l)
        cp.start()
        lin_copies.append(cp)
    for cp in lin_copies:
        cp.wait()

    bias_vec = bias_v[...]

    def finish(g, carry):
        b0 = g * 16
        t = jnp.zeros((16,), jnp.float32)
        for d in range(D):
            t = t + pv[pl.ds(d * BPW + b0, 16)]
        lsum = bias_vec
        for f in range(F):
            lsum = lsum + lin_v[pl.ds(f * BPW + b0, 16)]
        out_v[pl.ds(b0, 16)] = 0.5 * t + lsum
        return carry

    lax.fori_loop(0, BPW // 16, finish, 0)
    pltpu.sync_copy(out_v, out_hbm.at[pl.ds(w * BPW, BPW)])


_mesh = plsc.VectorSubcoreMesh(core_axis_name="c", subcore_axis_name="s")

_stream = functools.partial(
    pl.kernel,
    mesh=_mesh,
    compiler_params=pltpu.CompilerParams(
        needs_layout_passes=False, use_tc_tiling_on_sc=True),
    out_type=jax.ShapeDtypeStruct((D * B,), jnp.float32),
    scratch_types=[
        pltpu.VMEM((B,), jnp.int32),
        pltpu.VMEM((2 * HBUF,), jnp.float32),
        pltpu.VMEM((F * TAILW,), jnp.float32),
        pltpu.VMEM((B,), jnp.float32),
        pltpu.VMEM((B,), jnp.float32),
        pltpu.SemaphoreType.DMA,
        pltpu.SemaphoreType.DMA,
        pltpu.SemaphoreType.DMA,
    ],
)(_stream_body)

_reduce = functools.partial(
    pl.kernel,
    mesh=_mesh,
    compiler_params=pltpu.CompilerParams(
        needs_layout_passes=False, use_tc_tiling_on_sc=False),
    out_type=jax.ShapeDtypeStruct((B,), jnp.float32),
    scratch_types=[
        pltpu.VMEM((D * BPW,), jnp.float32),
        pltpu.VMEM((F * BPW,), jnp.int32),
        pltpu.VMEM((F * BPW,), jnp.int32),
        pltpu.VMEM((F * BPW,), jnp.float32),
        pltpu.VMEM((BPW,), jnp.float32),
        pltpu.VMEM((16,), jnp.float32),
        pltpu.SemaphoreType.DMA,
        pltpu.SemaphoreType.DMA,
    ],
)(_reduce_body)


@jax.jit
def kernel(Xi, W_lin, W_emb, bias):
    xi = Xi.reshape(B, F).astype(jnp.int32).T.reshape(F * B)
    emb = jnp.transpose(W_emb, (0, 2, 1)).reshape(F * D, VOCAB)
    tails = jnp.transpose(W_emb[:, VMAIN:, :], (0, 2, 1))
    tails = jnp.pad(tails, ((0, 0), (0, 0), (0, TAILW - (VOCAB - VMAIN))))
    tails = tails.reshape(F * D * TAILW)
    lin = W_lin.reshape(F * VOCAB)
    bias16 = jnp.broadcast_to(bias, (16,))
    parts = _stream(xi, emb, tails)
    return _reduce(parts, xi, lin, bias16)


# speedup vs baseline: 1.2110x; 1.0031x over previous
import functools
import jax
import jax.numpy as jnp
from jax import lax
from jax.experimental import pallas as pl
from jax.experimental.pallas import tpu as pltpu
from jax.experimental.pallas import tpu_sc as plsc

B = 4096
F = 26
VOCAB = 100000
D = 32

NC = 2
NS = 16
NW = NC * NS
BPW = B // NW
NG = B // 16

VMAIN = 99968
H0 = 50048
H1 = VMAIN - H0
HBUF = H0
TAILW = 128


def _stream_body(xi_hbm, emb_hbm, tails_hbm, parts_hbm,
                 idx_v, buf, tails_v, s_acc, q_acc, sem0, sem1, sem_t):
    c = lax.axis_index("c")
    s = lax.axis_index("s")
    w = s * NC + c

    zero = jnp.zeros((16,), jnp.float32)

    def clear(i, carry):
        s_acc[pl.ds(i * 16, 16)] = zero
        q_acc[pl.ds(i * 16, 16)] = zero
        return carry

    lax.fori_loop(0, NG, clear, 0)

    tail_copies = []
    for f in range(F):
        cp = pltpu.make_async_copy(
            tails_hbm.at[pl.ds((f * D + w) * TAILW, TAILW)],
            tails_v.at[pl.ds(f * TAILW, TAILW)], sem_t)
        tail_copies.append(cp)

    sems = (sem0, sem1)
    HALF = ((0, H0), (H0, H1))

    def fire(f, h):
        st, ln = HALF[h]
        pltpu.make_async_copy(
            emb_hbm.at[f * D + w, pl.ds(st, ln)],
            buf.at[pl.ds(h * HBUF, ln)], sems[h]).start()

    def drain(h):
        st, ln = HALF[h]
        pltpu.make_async_copy(
            emb_hbm.at[0, pl.ds(st, ln)],
            buf.at[pl.ds(h * HBUF, ln)], sems[h]).wait()

    def scan_half(f, h):
        base, hlen = HALF[h]
        boff = h * HBUF

        def scan(g, carry):
            p = pl.ds(g * 16, 16)
            v = idx_v[p]
            voff = v - base
            m = jnp.logical_and(voff >= 0, voff < hlen)
            vv = plsc.load_gather(
                buf, [boff + jnp.where(m, voff, 0)], mask=m)
            vsel = jnp.where(m, vv, 0.0)
            if h == 1:
                toff = v - VMAIN
                mt = toff >= 0
                tv = plsc.load_gather(
                    tails_v, [f * TAILW + jnp.where(mt, toff, 0)], mask=mt)
                vsel = vsel + jnp.where(mt, tv, 0.0)
            s_acc[p] = s_acc[p] + vsel
            q_acc[p] = q_acc[p] + vsel * vsel
            return carry

        lax.fori_loop(0, NG, scan, 0)

    fire(0, 0)

    def step(f, carry):
        pltpu.sync_copy(xi_hbm.at[pl.ds(f * B, B)], idx_v)
        fire(f, 1)
        drain(0)
        scan_half(f, 0)

        @pl.when(f + 1 < F)
        def _():
            fire(f + 1, 0)

        drain(1)
        scan_half(f, 1)
        return carry

    for cp in tail_copies:
        cp.start()
    for cp in tail_copies:
        cp.wait()
    lax.fori_loop(0, F, step, 0)

    def fin(g, carry):
        p = pl.ds(g * 16, 16)
        sa = s_acc[p]
        s_acc[p] = sa * sa - q_acc[p]
        return carry

    lax.fori_loop(0, NG, fin, 0)
    pltpu.sync_copy(s_acc, parts_hbm.at[pl.ds(w * B, B)])


def _reduce_body(parts_hbm, xi_hbm, lin_hbm, bias_hbm, out_hbm,
                 pv, idx_v, lidx, lin_v, out_v, bias_v, sem_p, sem_l):
    c = lax.axis_index("c")
    s = lax.axis_index("s")
    w = s * NC + c

    pltpu.sync_copy(bias_hbm, bias_v)

    stage = []
    for d in range(D):
        stage.append(pltpu.make_async_copy(
            parts_hbm.at[pl.ds(d * B + w * BPW, BPW)],
            pv.at[pl.ds(d * BPW, BPW)], sem_p))
    for f in range(F):
        stage.append(pltpu.make_async_copy(
            xi_hbm.at[pl.ds(f * B + w * BPW, BPW)],
            idx_v.at[pl.ds(f * BPW, BPW)], sem_p))
    for cp in stage:
        cp.start()
    for cp in stage:
        cp.wait()

    def lin_build(f, carry):
        def per_stripe(st, carry2):
            p = pl.ds(f * BPW + st * 16, 16)
            lidx[p] = f * VOCAB + idx_v[p]
            return carry2
        lax.fori_loop(0, BPW // 16, per_stripe, 0)
        return carry

    lax.fori_loop(0, F, lin_build, 0)
    lin_copies = []
    for j in range(F):
        cp = pltpu.make_async_copy(
            lin_hbm.at[lidx.at[pl.ds(j * 128, 128)]],
            lin_v.at[pl.ds(j * 128, 128)], sem_l)
        cp.start()
        lin_copies.append(cp)
    for cp in lin_copies:
        cp.wait()

    bias_vec = bias_v[...]

    def finish(g, carry):
        b0 = g * 16
        t = jnp.zeros((16,), jnp.float32)
        for d in range(D):
            t = t + pv[pl.ds(d * BPW + b0, 16)]
        lsum = bias_vec
        for f in range(F):
            lsum = lsum + lin_v[pl.ds(f * BPW + b0, 16)]
        out_v[pl.ds(b0, 16)] = 0.5 * t + lsum
        return carry

    lax.fori_loop(0, BPW // 16, finish, 0)
    pltpu.sync_copy(out_v, out_hbm.at[pl.ds(w * BPW, BPW)])


_mesh = plsc.VectorSubcoreMesh(core_axis_name="c", subcore_axis_name="s")

_stream = functools.partial(
    pl.kernel,
    mesh=_mesh,
    compiler_params=pltpu.CompilerParams(
        needs_layout_passes=False, use_tc_tiling_on_sc=True),
    out_type=jax.ShapeDtypeStruct((D * B,), jnp.float32),
    scratch_types=[
        pltpu.VMEM((B,), jnp.int32),
        pltpu.VMEM((2 * HBUF,), jnp.float32),
        pltpu.VMEM((F * TAILW,), jnp.float32),
        pltpu.VMEM((B,), jnp.float32),
        pltpu.VMEM((B,), jnp.float32),
        pltpu.SemaphoreType.DMA,
        pltpu.SemaphoreType.DMA,
        pltpu.SemaphoreType.DMA,
    ],
)(_stream_body)

_reduce = functools.partial(
    pl.kernel,
    mesh=_mesh,
    compiler_params=pltpu.CompilerParams(
        needs_layout_passes=False, use_tc_tiling_on_sc=False),
    out_type=jax.ShapeDtypeStruct((B,), jnp.float32),
    scratch_types=[
        pltpu.VMEM((D * BPW,), jnp.float32),
        pltpu.VMEM((F * BPW,), jnp.int32),
        pltpu.VMEM((F * BPW,), jnp.int32),
        pltpu.VMEM((F * BPW,), jnp.float32),
        pltpu.VMEM((BPW,), jnp.float32),
        pltpu.VMEM((16,), jnp.float32),
        pltpu.SemaphoreType.DMA,
        pltpu.SemaphoreType.DMA,
    ],
)(_reduce_body)


@jax.jit
def kernel(Xi, W_lin, W_emb, bias):
    xi = Xi.reshape(B, F).astype(jnp.int32).T.reshape(F * B)
    emb = jnp.transpose(W_emb, (0, 2, 1)).reshape(F * D, VOCAB)
    tails = jnp.transpose(W_emb[:, VMAIN:, :], (0, 2, 1))
    tails = jnp.pad(tails, ((0, 0), (0, 0), (0, TAILW - (VOCAB - VMAIN))))
    tails = tails.reshape(F * D * TAILW)
    lin = W_lin.reshape(F * VOCAB)
    bias16 = jnp.broadcast_to(bias, (16,))
    parts = _stream(xi, emb, tails)
    return _reduce(parts, xi, lin, bias16)
